# baseline (speedup 1.0000x reference)
import functools

import jax
import jax.numpy as jnp
from jax import lax
from jax.experimental import pallas as pl
from jax.experimental.pallas import tpu as pltpu
from jax.experimental.pallas import tpu_sc as plsc

_LARGE_NEG = -100000.0
_BSZ, _NP, _NL = 128, 200, 48
_PP = 208
_NW = 32
_RPW = _BSZ // _NW
_NEG = -3.0e38
_LN2 = 0.6931471805599453


_GDN = lax.GatherDimensionNumbers(
    offset_dims=(), collapsed_slice_dims=(0,), start_index_map=(0,))


def _perm(x, idx):
    return lax.gather(x, idx[:, None], _GDN, (1,),
                      mode=lax.GatherScatterMode.PROMISE_IN_BOUNDS)


def _bfly(x, op, lane):
    for d in (8, 4, 2, 1):
        x = op(x, _perm(x, lane ^ d))
    return x


@functools.partial(
    pl.kernel,
    out_type=[
        jax.ShapeDtypeStruct((_NW, 128), jnp.int32),
        jax.ShapeDtypeStruct((_NW, 128), jnp.float32),
    ],
    mesh=plsc.VectorSubcoreMesh(core_axis_name="c", subcore_axis_name="s"),
    scratch_types=[
        pltpu.VMEM((_RPW, _PP), jnp.int32),
        pltpu.VMEM((_RPW, _PP), jnp.float32),
        pltpu.VMEM((_RPW, _PP), jnp.float32),
        pltpu.VMEM((2 * _RPW, 128), jnp.int32),
        pltpu.VMEM((2 * _RPW, 128), jnp.float32),
        pltpu.VMEM((_RPW, _PP), jnp.float32),
        pltpu.VMEM((128,), jnp.int32),
        pltpu.VMEM((128,), jnp.float32),
        pltpu.SemaphoreType.DMA,
    ],
)
def _sched(flat_hbm, li_hbm, mk_hbm, gm_hbm, act_hbm, lp_hbm,
           li_v, mk_v, gm_v, idx_v, gat_v, prio_v, outa_v, outl_v, sem):
    wid = lax.axis_index("s") * 2 + lax.axis_index("c")
    row0 = wid * _RPW
    pltpu.sync_copy(li_hbm.at[pl.ds(row0, _RPW)], li_v)
    pltpu.sync_copy(mk_hbm.at[pl.ds(row0, _RPW)], mk_v)
    pltpu.sync_copy(gm_hbm.at[pl.ds(row0, _RPW)], gm_v)

    lane = lax.iota(jnp.int32, 16)

    for r in range(_RPW):
        rowbase = (row0 + r) * (_NP * _NL)
        for j in range(2):
            for t in range(8):
                p0 = j * 128 + t * 16
                if p0 >= _PP:
                    idx_v[2 * r + j, pl.ds(t * 16, 16)] = jnp.zeros((16,), jnp.int32)
                    continue
                pv = lane + p0
                liv = li_v[r, pl.ds(p0, 16)]
                fi = rowbase + pv * _NL + liv
                idx_v[2 * r + j, pl.ds(t * 16, 16)] = jnp.where(pv < _NP, fi, 0)

    handles = [
        pltpu.async_copy(flat_hbm.at[idx_v.at[k]], gat_v.at[k], sem)
        for k in range(2 * _RPW)
    ]
    for h in handles:
        h.wait()

    acc_a = jnp.zeros((16,), jnp.int32)
    acc_l = jnp.zeros((16,), jnp.float32)
    for r in range(_RPW):
        pmax = jnp.full((16,), _NEG, jnp.float32)
        bz = jnp.full((16,), _NEG, jnp.float32)
        bc = jnp.zeros((16,), jnp.int32)
        for c in range(_PP // 16):
            k = 2 * r + (1 if c >= 8 else 0)
            off = (c % 8) * 16
            gv = gat_v[k, pl.ds(off, 16)]
            mv = mk_v[r, pl.ds(c * 16, 16)]
            gb = gm_v[r, pl.ds(c * 16, 16)]
            prio = gv + (1.0 - mv) * _LARGE_NEG
            prio_v[r, pl.ds(c * 16, 16)] = prio
            pvld = (lane + c * 16) < _NP
            pmax = jnp.maximum(pmax, jnp.where(pvld, prio, _NEG))
            z = jnp.where(pvld, prio + gb, _NEG)
            upd = z > bz
            bc = jnp.where(upd, c, bc)
            bz = jnp.where(upd, z, bz)
        zmax = _bfly(bz, jnp.maximum, lane)
        cand = jnp.where(bz == zmax, bc * 16 + lane, 1 << 30)
        p_star = _bfly(cand, jnp.minimum, lane)
        pmaxs = _bfly(pmax, jnp.maximum, lane)
        sv = jnp.zeros((16,), jnp.float32)
        pa = jnp.full((16,), _NEG, jnp.float32)
        for c in range(_PP // 16):
            prio = prio_v[r, pl.ds(c * 16, 16)]
            pvld = (lane + c * 16) < _NP
            sv = sv + jnp.where(pvld, jnp.exp(prio - pmaxs), 0.0)
            pa = jnp.where((lane + c * 16) == p_star, prio, pa)
        pa = _bfly(pa, jnp.maximum, lane)
        vs = _bfly(sv, jnp.add, lane)
        ebits = lax.shift_right_arithmetic(
            lax.bitcast_convert_type(vs, jnp.int32), 23) - 127
        y = ebits.astype(jnp.float32) * _LN2 + 0.375
        for _ in range(4):
            y = y + vs * jnp.exp(-y) - 1.0
        lp = pa - pmaxs - y
        is_r = lane == r
        acc_a = jnp.where(is_r, p_star, acc_a)
        acc_l = jnp.where(is_r, lp, acc_l)

    for t in range(8):
        outa_v[pl.ds(t * 16, 16)] = acc_a if t == 0 else jnp.zeros((16,), jnp.int32)
        outl_v[pl.ds(t * 16, 16)] = acc_l if t == 0 else jnp.zeros((16,), jnp.float32)
    pltpu.sync_copy(outa_v, act_hbm.at[wid])
    pltpu.sync_copy(outl_v, lp_hbm.at[wid])


def kernel(all_has_answer_logits, layer_indices, masks, init_priorities):
    del init_priorities
    bsz, npass, _ = all_has_answer_logits.shape
    pad = _PP - npass
    flat = all_has_answer_logits.reshape(-1)
    li = jnp.pad(layer_indices, ((0, 0), (0, pad)))
    mk = jnp.pad(masks, ((0, 0), (0, pad)))
    gm = jax.random.gumbel(jax.random.key(42), (bsz, npass), jnp.float32)
    gm = jnp.pad(gm, ((0, 0), (0, pad)))
    act2, lp2 = _sched(flat, li, mk, gm)
    return (act2[:, :_RPW].reshape(bsz), lp2[:, :_RPW].reshape(bsz))


# speedup vs baseline: 1.0108x; 1.0108x over previous
import functools

import jax
import jax.numpy as jnp
from jax import lax
from jax.experimental import pallas as pl
from jax.experimental.pallas import tpu as pltpu
from jax.experimental.pallas import tpu_sc as plsc

_LARGE_NEG = -100000.0
_BSZ, _NP, _NL = 128, 200, 48
_NW = 32
_RPW = _BSZ // _NW
_BPR = 2
_NBR = _RPW * _BPR
_NCH = 13
_NEG = -3.0e38
_LN2 = 0.6931471805599453

_GDN = lax.GatherDimensionNumbers(
    offset_dims=(), collapsed_slice_dims=(0,), start_index_map=(0,))


def _perm(x, idx):
    return lax.gather(x, idx[:, None], _GDN, (1,),
                      mode=lax.GatherScatterMode.PROMISE_IN_BOUNDS)


def _bfly(x, op, lane):
    for d in (8, 4, 2, 1):
        x = op(x, _perm(x, lane ^ d))
    return x


@functools.partial(
    pl.kernel,
    out_type=[
        jax.ShapeDtypeStruct((_NW, 128), jnp.int32),
        jax.ShapeDtypeStruct((_NW, 128), jnp.float32),
    ],
    mesh=plsc.VectorSubcoreMesh(core_axis_name="c", subcore_axis_name="s"),
    scratch_types=[
        pltpu.VMEM((_NBR, 128), jnp.int32),
        pltpu.VMEM((_NBR, 128), jnp.float32),
        pltpu.VMEM((_NBR, 128), jnp.float32),
        pltpu.VMEM((_NBR, 128), jnp.int32),
        pltpu.VMEM((_NBR, 128), jnp.float32),
        pltpu.VMEM((128,), jnp.int32),
        pltpu.VMEM((128,), jnp.float32),
        pltpu.SemaphoreType.DMA,
        pltpu.SemaphoreType.DMA,
    ],
)
def _sched(flat_hbm, li_hbm, mk_hbm, gm_hbm, act_hbm, lp_hbm,
           li_v, mk_v, gm_v, idx_v, gat_v, outa_v, outl_v, sem, sem2):
    wid = lax.axis_index("s") * 2 + lax.axis_index("c")
    row0 = wid * _RPW
    lane = lax.iota(jnp.int32, 16)

    pltpu.sync_copy(li_hbm.at[wid], li_v)
    bg_mk = pltpu.async_copy(mk_hbm.at[wid], mk_v, sem2)
    bg_gm = pltpu.async_copy(gm_hbm.at[wid], gm_v, sem2)

    lane48 = lane * _NL
    for r in range(_RPW):
        rowbase = (row0 + r) * (_NP * _NL)
        for k in range(_BPR):
            for t in range(8):
                p0 = k * 128 + t * 16
                liv = li_v[_BPR * r + k, pl.ds(t * 16, 16)]
                if p0 + 16 <= _NP:
                    fi = (rowbase + p0 * _NL) + lane48 + liv
                elif p0 < _NP:
                    pv = lane + p0
                    fi = jnp.where(pv < _NP,
                                   (rowbase + p0 * _NL) + lane48 + liv, 0)
                else:
                    fi = jnp.zeros((16,), jnp.int32)
                idx_v[_BPR * r + k, pl.ds(t * 16, 16)] = fi

    handles = [
        pltpu.async_copy(flat_hbm.at[idx_v.at[k]], gat_v.at[k], sem)
        for k in range(_NBR)
    ]
    bg_mk.wait()
    bg_gm.wait()
    for h in handles:
        h.wait()

    acc_a = jnp.zeros((16,), jnp.int32)
    acc_l = jnp.zeros((16,), jnp.float32)
    for r in range(_RPW):
        pmax = jnp.full((16,), _NEG, jnp.float32)
        bz = jnp.full((16,), _NEG, jnp.float32)
        bc = jnp.zeros((16,), jnp.int32)
        prios = []
        for c in range(_NCH):
            k = _BPR * r + (1 if c >= 8 else 0)
            off = (c % 8) * 16
            gv = gat_v[k, pl.ds(off, 16)]
            mv = mk_v[k, pl.ds(off, 16)]
            gb = gm_v[k, pl.ds(off, 16)]
            prio = gv + (1.0 - mv) * _LARGE_NEG
            if c == _NCH - 1:
                prio = jnp.where(lane + c * 16 < _NP, prio, _NEG)
            prios.append(prio)
            pmax = jnp.maximum(pmax, prio)
            z = prio + gb
            upd = z > bz
            bc = jnp.where(upd, c, bc)
            bz = jnp.where(upd, z, bz)
        zmax = _bfly(bz, jnp.maximum, lane)
        cand = jnp.where(bz == zmax, bc * 16 + lane, 1 << 30)
        p_star = _bfly(cand, jnp.minimum, lane)
        pmaxs = _bfly(pmax, jnp.maximum, lane)
        sv = jnp.zeros((16,), jnp.float32)
        pa = jnp.full((16,), _NEG, jnp.float32)
        for c in range(_NCH):
            sv = sv + jnp.exp(prios[c] - pmaxs)
            pa = jnp.where((lane + c * 16) == p_star, prios[c], pa)
        pa = _bfly(pa, jnp.maximum, lane)
        vs = _bfly(sv, jnp.add, lane)
        ebits = lax.shift_right_arithmetic(
            lax.bitcast_convert_type(vs, jnp.int32), 23) - 127
        y = ebits.astype(jnp.float32) * _LN2 + 0.375
        for _ in range(4):
            y = y + vs * jnp.exp(-y) - 1.0
        lp = pa - pmaxs - y
        is_r = lane == r
        acc_a = jnp.where(is_r, p_star, acc_a)
        acc_l = jnp.where(is_r, lp, acc_l)

    for t in range(8):
        outa_v[pl.ds(t * 16, 16)] = acc_a if t == 0 else jnp.zeros((16,), jnp.int32)
        outl_v[pl.ds(t * 16, 16)] = acc_l if t == 0 else jnp.zeros((16,), jnp.float32)
    pltpu.sync_copy(outa_v, act_hbm.at[wid])
    pltpu.sync_copy(outl_v, lp_hbm.at[wid])


def _to_worker_layout(x):
    pad = jnp.pad(x, ((0, 0), (0, 256 - _NP)))
    return pad.reshape(_NW, _NBR, 128)


def kernel(all_has_answer_logits, layer_indices, masks, init_priorities):
    del init_priorities
    bsz, npass, _ = all_has_answer_logits.shape
    flat = all_has_answer_logits.reshape(-1)
    li = _to_worker_layout(layer_indices)
    mk = _to_worker_layout(masks)
    gm = _to_worker_layout(
        jax.random.gumbel(jax.random.key(42), (bsz, npass), jnp.float32))
    act2, lp2 = _sched(flat, li, mk, gm)
    return (act2[:, :_RPW].reshape(bsz), lp2[:, :_RPW].reshape(bsz))


# speedup vs baseline: 1.6362x; 1.6187x over previous
import functools

import jax
import jax.numpy as jnp
from jax import lax
from jax.experimental import pallas as pl
from jax.experimental.pallas import tpu as pltpu
from jax.experimental.pallas import tpu_sc as plsc

_LARGE_NEG = -100000.0
_BSZ, _NP, _NL = 128, 200, 48
_NW = 32
_RPW = _BSZ // _NW
_BPR = 2
_NBR = _RPW * _BPR
_NCH = 13
_NEG = -3.0e38
_LN2 = 0.6931471805599453

_GDN = lax.GatherDimensionNumbers(
    offset_dims=(), collapsed_slice_dims=(0,), start_index_map=(0,))


def _perm(x, idx):
    return lax.gather(x, idx[:, None], _GDN, (1,),
                      mode=lax.GatherScatterMode.PROMISE_IN_BOUNDS)


def _bfly(x, op, lane):
    for d in (8, 4, 2, 1):
        x = op(x, _perm(x, lane ^ d))
    return x


@functools.partial(
    pl.kernel,
    out_type=[
        jax.ShapeDtypeStruct((_NW, 128), jnp.int32),
        jax.ShapeDtypeStruct((_NW, 128), jnp.float32),
    ],
    mesh=plsc.VectorSubcoreMesh(core_axis_name="c", subcore_axis_name="s"),
    scratch_types=[
        pltpu.VMEM((_NBR, 128), jnp.int32),
        pltpu.VMEM((_NBR, 128), jnp.float32),
        pltpu.VMEM((_NBR, 128), jnp.float32),
        pltpu.VMEM((_NBR, 128), jnp.int32),
        pltpu.VMEM((_NBR, 128), jnp.float32),
        pltpu.VMEM((128,), jnp.int32),
        pltpu.VMEM((128,), jnp.float32),
        pltpu.SemaphoreType.DMA,
        pltpu.SemaphoreType.DMA,
    ],
)
def _sched(flat_hbm, li_hbm, mk_hbm, gm_hbm, act_hbm, lp_hbm,
           li_v, mk_v, gm_v, idx_v, gat_v, outa_v, outl_v, sem, sem2):
    wid = lax.axis_index("s") * 2 + lax.axis_index("c")
    row0 = wid * _RPW
    lane = lax.iota(jnp.int32, 16)

    pltpu.sync_copy(li_hbm.at[wid], li_v)
    bg_mk = pltpu.async_copy(mk_hbm.at[wid], mk_v, sem2)
    bg_gm = pltpu.async_copy(gm_hbm.at[wid], gm_v, sem2)

    lane48 = lane * _NL
    for r in range(_RPW):
        rowbase = (row0 + r) * (_NP * _NL)
        for k in range(_BPR):
            for t in range(8):
                p0 = k * 128 + t * 16
                liv = li_v[_BPR * r + k, pl.ds(t * 16, 16)]
                if p0 + 16 <= _NP:
                    fi = (rowbase + p0 * _NL) + lane48 + liv
                elif p0 < _NP:
                    pv = lane + p0
                    fi = jnp.where(pv < _NP,
                                   (rowbase + p0 * _NL) + lane48 + liv,
                                   (rowbase + (p0 - _NP) * _NL) + lane48)
                else:
                    fi = (rowbase + (p0 - _NP) * _NL) + lane48
                idx_v[_BPR * r + k, pl.ds(t * 16, 16)] = fi

    handles = [
        pltpu.async_copy(flat_hbm.at[idx_v.at[k]], gat_v.at[k], sem)
        for k in range(_NBR)
    ]
    bg_mk.wait()
    bg_gm.wait()
    for h in handles:
        h.wait()

    acc_a = jnp.zeros((16,), jnp.int32)
    acc_l = jnp.zeros((16,), jnp.float32)
    for r in range(_RPW):
        pmax = jnp.full((16,), _NEG, jnp.float32)
        bz = jnp.full((16,), _NEG, jnp.float32)
        bc = jnp.zeros((16,), jnp.int32)
        prios = []
        for c in range(_NCH):
            k = _BPR * r + (1 if c >= 8 else 0)
            off = (c % 8) * 16
            gv = gat_v[k, pl.ds(off, 16)]
            mv = mk_v[k, pl.ds(off, 16)]
            gb = gm_v[k, pl.ds(off, 16)]
            prio = gv + (1.0 - mv) * _LARGE_NEG
            if c == _NCH - 1:
                prio = jnp.where(lane + c * 16 < _NP, prio, _NEG)
            prios.append(prio)
            pmax = jnp.maximum(pmax, prio)
            z = prio + gb
            upd = z > bz
            bc = jnp.where(upd, c, bc)
            bz = jnp.where(upd, z, bz)
        zmax = _bfly(bz, jnp.maximum, lane)
        cand = jnp.where(bz == zmax, bc * 16 + lane, 1 << 30)
        p_star = _bfly(cand, jnp.minimum, lane)
        pmaxs = _bfly(pmax, jnp.maximum, lane)
        sv = jnp.zeros((16,), jnp.float32)
        pa = jnp.full((16,), _NEG, jnp.float32)
        for c in range(_NCH):
            sv = sv + jnp.exp(prios[c] - pmaxs)
            pa = jnp.where((lane + c * 16) == p_star, prios[c], pa)
        pa = _bfly(pa, jnp.maximum, lane)
        vs = _bfly(sv, jnp.add, lane)
        ebits = lax.shift_right_arithmetic(
            lax.bitcast_convert_type(vs, jnp.int32), 23) - 127
        y = ebits.astype(jnp.float32) * _LN2 + 0.375
        for _ in range(4):
            y = y + vs * jnp.exp(-y) - 1.0
        lp = pa - pmaxs - y
        is_r = lane == r
        acc_a = jnp.where(is_r, p_star, acc_a)
        acc_l = jnp.where(is_r, lp, acc_l)

    for t in range(8):
        outa_v[pl.ds(t * 16, 16)] = acc_a if t == 0 else jnp.zeros((16,), jnp.int32)
        outl_v[pl.ds(t * 16, 16)] = acc_l if t == 0 else jnp.zeros((16,), jnp.float32)
    pltpu.sync_copy(outa_v, act_hbm.at[wid])
    pltpu.sync_copy(outl_v, lp_hbm.at[wid])


def _to_worker_layout(x):
    pad = jnp.pad(x, ((0, 0), (0, 256 - _NP)))
    return pad.reshape(_NW, _NBR, 128)


def kernel(all_has_answer_logits, layer_indices, masks, init_priorities):
    del init_priorities
    bsz, npass, _ = all_has_answer_logits.shape
    flat = all_has_answer_logits.reshape(-1)
    li = _to_worker_layout(layer_indices)
    mk = _to_worker_layout(masks)
    gm = _to_worker_layout(
        jax.random.gumbel(jax.random.key(42), (bsz, npass), jnp.float32))
    act2, lp2 = _sched(flat, li, mk, gm)
    return (act2[:, :_RPW].reshape(bsz), lp2[:, :_RPW].reshape(bsz))
